# baseline (speedup 1.0000x reference)
import functools

import jax
import jax.numpy as jnp
from jax import lax
from jax.experimental import pallas as pl
from jax.experimental.pallas import tpu as pltpu
from jax.experimental.pallas import tpu_sc as plsc

N = 10000
D = 128
H = 64
G = 64

NC = 2
NS = 16
NW = NC * NS

B = 128
N_PAD = NS * 640
ROWS_PT = N_PAD // NS

_mesh = plsc.VectorSubcoreMesh(core_axis_name="c", subcore_axis_name="s")
_sc_params = pltpu.CompilerParams(use_tc_tiling_on_sc=False)


def _edge_layout(E):
    q = B * NBUF
    per_w = ((E + NW - 1) // NW + q - 1) // q * q
    return per_w, per_w // B, NW * per_w


NBUF = 4


@functools.lru_cache(maxsize=None)
def _make_deg_kernel(E):
    PER_W, CHUNKS, _ = _edge_layout(E)

    @functools.partial(
        pl.kernel,
        out_type=jax.ShapeDtypeStruct((NC, N_PAD, 8), jnp.float32),
        mesh=_mesh,
        scratch_types=[
            pltpu.VMEM_SHARED((N_PAD, 8), jnp.float32),
            pltpu.VMEM((CHUNKS, B), jnp.int32),
            pltpu.VMEM((B, 8), jnp.float32),
            pltpu.SemaphoreType.DMA((NBUF,)),
        ],
        compiler_params=_sc_params,
    )
    def deg_kernel(dst_hbm, zeros_hbm, ones_hbm, out_hbm, accum, dstbuf, ones_v, sems):
        cid = lax.axis_index("c")
        sid = lax.axis_index("s")
        wid = cid * NS + sid
        pltpu.sync_copy(dst_hbm.at[wid], dstbuf)
        pltpu.sync_copy(zeros_hbm.at[pl.ds(0, ROWS_PT)], accum.at[pl.ds(sid * ROWS_PT, ROWS_PT)])
        pltpu.sync_copy(ones_hbm, ones_v)
        plsc.subcore_barrier()

        def group(g, carry):
            for b in range(NBUF):
                pltpu.async_copy(ones_v, accum.at[dstbuf.at[g * NBUF + b]],
                                 sems.at[b], add=True)
            for b in range(NBUF):
                pltpu.make_async_copy(ones_v, accum.at[dstbuf.at[g * NBUF + b]],
                                      sems.at[b]).wait()
            return carry

        lax.fori_loop(0, CHUNKS // NBUF, group, 0)
        plsc.subcore_barrier()
        r0 = sid * ROWS_PT
        pltpu.sync_copy(accum.at[pl.ds(r0, ROWS_PT)], out_hbm.at[cid, pl.ds(r0, ROWS_PT)])

    return deg_kernel


@functools.lru_cache(maxsize=None)
def _make_agg_kernel(E, F):
    PER_W, _, _ = _edge_layout(E)
    BC = 8192 // F
    CHUNKS = PER_W // BC
    nbuf = 4 if F <= 64 else 2
    assert CHUNKS % nbuf == 0 and CHUNKS // nbuf >= 2

    @functools.partial(
        pl.kernel,
        out_type=jax.ShapeDtypeStruct((NC, N_PAD, F), jnp.float32),
        mesh=_mesh,
        scratch_types=[
            pltpu.VMEM_SHARED((N_PAD, F), jnp.float32),
            pltpu.VMEM((CHUNKS, BC), jnp.int32),
            pltpu.VMEM((CHUNKS, BC), jnp.int32),
            pltpu.VMEM((nbuf, BC, F), jnp.float32),
            pltpu.SemaphoreType.DMA((nbuf,)),
            pltpu.SemaphoreType.DMA((nbuf,)),
        ],
        compiler_params=_sc_params,
    )
    def agg_kernel(src_hbm, dst_hbm, z_hbm, zeros_hbm, out_hbm,
                   accum, srcbuf, dstbuf, rows, gsem, ssem):
        cid = lax.axis_index("c")
        sid = lax.axis_index("s")
        wid = cid * NS + sid
        pltpu.sync_copy(src_hbm.at[wid], srcbuf)
        pltpu.sync_copy(dst_hbm.at[wid], dstbuf)

        def gather(i, b):
            pltpu.async_copy(z_hbm.at[srcbuf.at[i]], rows.at[b], gsem.at[b])

        def gather_wait(b):
            pltpu.make_async_copy(z_hbm.at[srcbuf.at[0]], rows.at[b], gsem.at[b]).wait()

        def scatter(i, b):
            pltpu.async_copy(rows.at[b], accum.at[dstbuf.at[i]], ssem.at[b], add=True)

        def scatter_wait(i, b):
            pltpu.make_async_copy(rows.at[b], accum.at[dstbuf.at[i]], ssem.at[b]).wait()

        for b in range(nbuf):
            gather(b, b)
        pltpu.sync_copy(zeros_hbm.at[pl.ds(0, ROWS_PT)], accum.at[pl.ds(sid * ROWS_PT, ROWS_PT)])
        plsc.subcore_barrier()

        def group(g, carry):
            i0 = g * nbuf
            for b in range(nbuf):
                gather_wait(b)
                scatter(i0 + b, b)
            for b in range(nbuf):
                scatter_wait(i0 + b, b)
                gather(i0 + nbuf + b, b)
            return carry

        lax.fori_loop(0, CHUNKS // nbuf - 1, group, 0)
        i0 = CHUNKS - nbuf
        for b in range(nbuf):
            gather_wait(b)
            scatter(i0 + b, b)
        for b in range(nbuf):
            scatter_wait(i0 + b, b)
        plsc.subcore_barrier()
        r0 = sid * ROWS_PT
        pltpu.sync_copy(accum.at[pl.ds(r0, ROWS_PT)], out_hbm.at[cid, pl.ds(r0, ROWS_PT)])

    return agg_kernel


def _tc1_body(degA, degB, x, W1, dis_o, z1_o):
    dis = lax.rsqrt(degA[...] + degB[...] + 1.0)
    dis_o[...] = dis
    z1_o[...] = dis * jnp.dot(x[...], W1[...], preferred_element_type=jnp.float32)


def _tc2_body(pA, pB, z1, dis, b1, z2_o):
    d = dis[...]
    a = d * (pA[...] + pB[...] + z1[...])
    z2_o[...] = d * jnp.maximum(a + b1[...], 0.0)


def _tc3_body(pA, pB, z2, dis, W2, b2, z3_o):
    d = dis[...]
    a = d * (pA[...] + pB[...] + z2[...])
    h = jnp.maximum(jnp.dot(a, W2[...], preferred_element_type=jnp.float32) + b2[...], 0.0)
    z3_o[...] = d * h


def _tc4_body(pA, pB, z3, dis, batch2, W3, b3, Wl, bl, out_o):
    a3 = dis[...] * (pA[...] + pB[...] + z3[...])
    seg = lax.broadcasted_iota(jnp.int32, (G, N), 0)
    onehot = (batch2[...] == seg).astype(jnp.float32)
    sums = jnp.dot(onehot, a3, preferred_element_type=jnp.float32)
    cnt = jnp.sum(onehot, axis=1, keepdims=True)
    pooled = sums / jnp.maximum(cnt, 1.0)
    head = jnp.dot(pooled, W3[...], preferred_element_type=jnp.float32) + b3[...]
    out_o[...] = jnp.dot(head, Wl[...], preferred_element_type=jnp.float32) + bl[...]


def _tc_call(body, out_shapes):
    return pl.pallas_call(body, out_shape=out_shapes)


def kernel(x, edge_index, batch, W1, b1, W2, b2, W3, b3, Wl, bl):
    E = edge_index.shape[1]

    PER_W, CHUNKS, E_PAD = _edge_layout(E)
    pad = E_PAD - E
    src_f = jnp.concatenate([edge_index[0], jnp.zeros((pad,), jnp.int32)])
    dst_f = jnp.concatenate([edge_index[1], jnp.full((pad,), N, jnp.int32)])
    src_c128 = src_f.reshape(NW, PER_W // 128, 128)
    dst_c128 = dst_f.reshape(NW, PER_W // 128, 128)
    src_c64 = src_f.reshape(NW, PER_W // 64, 64)
    dst_c64 = dst_f.reshape(NW, PER_W // 64, 64)

    zeros8 = jnp.zeros((ROWS_PT, 8), jnp.float32)
    ones8 = jnp.ones((B, 8), jnp.float32)
    zeros64 = jnp.zeros((ROWS_PT, H), jnp.float32)
    zeros128 = jnp.zeros((ROWS_PT, 2 * H), jnp.float32)

    deg_parts = _make_deg_kernel(E)(dst_c128, zeros8, ones8)
    degA = deg_parts[0, :N, 0:1]
    degB = deg_parts[1, :N, 0:1]

    agg64 = _make_agg_kernel(E, H)
    agg128 = _make_agg_kernel(E, 2 * H)

    dis, z1 = _tc_call(_tc1_body, (
        jax.ShapeDtypeStruct((N, 1), jnp.float32),
        jax.ShapeDtypeStruct((N, H), jnp.float32),
    ))(degA, degB, x, W1)

    s1 = agg64(src_c128, dst_c128, z1, zeros64)
    z2 = _tc_call(_tc2_body, jax.ShapeDtypeStruct((N, H), jnp.float32))(
        s1[0, :N, :], s1[1, :N, :], z1, dis, b1.reshape(1, H))

    s2 = agg64(src_c128, dst_c128, z2, zeros64)
    z3 = _tc_call(_tc3_body, jax.ShapeDtypeStruct((N, 2 * H), jnp.float32))(
        s2[0, :N, :], s2[1, :N, :], z2, dis, W2, b2.reshape(1, 2 * H))

    s3 = agg128(src_c64, dst_c64, z3, zeros128)
    out = _tc_call(_tc4_body, jax.ShapeDtypeStruct((G, 1), jnp.float32))(
        s3[0, :N, :], s3[1, :N, :], z3, dis, batch.reshape(1, N),
        W3, b3.reshape(1, 2 * H), Wl, bl.reshape(1, 1))
    return out


# speedup vs baseline: 1.0896x; 1.0896x over previous
import functools

import jax
import jax.numpy as jnp
from jax import lax
from jax.experimental import pallas as pl
from jax.experimental.pallas import tpu as pltpu
from jax.experimental.pallas import tpu_sc as plsc

N = 10000
D = 128
H = 64
G = 64

NC = 2
NS = 16
NW = NC * NS

B = 128
N_PAD = NS * 640
ROWS_PT = N_PAD // NS

_mesh = plsc.VectorSubcoreMesh(core_axis_name="c", subcore_axis_name="s")
_sc_params = pltpu.CompilerParams(use_tc_tiling_on_sc=False)


def _edge_layout(E):
    q = B * NBUF
    per_w = ((E + NW - 1) // NW + q - 1) // q * q
    return per_w, per_w // B, NW * per_w


NBUF = 4


@functools.lru_cache(maxsize=None)
def _make_deg_kernel(E):
    PER_W, CHUNKS, _ = _edge_layout(E)

    @functools.partial(
        pl.kernel,
        out_type=jax.ShapeDtypeStruct((NC, N_PAD, 8), jnp.float32),
        mesh=_mesh,
        scratch_types=[
            pltpu.VMEM_SHARED((N_PAD, 8), jnp.float32),
            pltpu.VMEM((CHUNKS, B), jnp.int32),
            pltpu.VMEM((B, 8), jnp.float32),
            pltpu.SemaphoreType.DMA((NBUF,)),
        ],
        compiler_params=_sc_params,
    )
    def deg_kernel(dst_hbm, zeros_hbm, ones_hbm, out_hbm, accum, dstbuf, ones_v, sems):
        cid = lax.axis_index("c")
        sid = lax.axis_index("s")
        wid = cid * NS + sid
        pltpu.sync_copy(dst_hbm.at[wid], dstbuf)
        pltpu.sync_copy(zeros_hbm.at[pl.ds(0, ROWS_PT)], accum.at[pl.ds(sid * ROWS_PT, ROWS_PT)])
        pltpu.sync_copy(ones_hbm, ones_v)
        plsc.subcore_barrier()

        def group(g, carry):
            for b in range(NBUF):
                pltpu.async_copy(ones_v, accum.at[dstbuf.at[g * NBUF + b]],
                                 sems.at[b], add=True)
            for b in range(NBUF):
                pltpu.make_async_copy(ones_v, accum.at[dstbuf.at[g * NBUF + b]],
                                      sems.at[b]).wait()
            return carry

        lax.fori_loop(0, CHUNKS // NBUF, group, 0)
        plsc.subcore_barrier()
        r0 = sid * ROWS_PT
        pltpu.sync_copy(accum.at[pl.ds(r0, ROWS_PT)], out_hbm.at[cid, pl.ds(r0, ROWS_PT)])

    return deg_kernel


@functools.lru_cache(maxsize=None)
def _make_agg_kernel(E, F, C0, C1):
    PER_W, _, _ = _edge_layout(E)
    BC = 8192 // F
    TOT_CHUNKS = NW * PER_W // BC
    nbuf = 4 if F <= 64 else 2
    assert NS * (C0 + C1) == TOT_CHUNKS
    assert C0 % nbuf == 0 and C1 % nbuf == 0 and C1 // nbuf >= 2

    @functools.partial(
        pl.kernel,
        out_type=jax.ShapeDtypeStruct((NC, N_PAD, F), jnp.float32),
        mesh=_mesh,
        scratch_types=[
            pltpu.VMEM_SHARED((N_PAD, F), jnp.float32),
            pltpu.VMEM((C0, BC), jnp.int32),
            pltpu.VMEM((C0, BC), jnp.int32),
            pltpu.VMEM((nbuf, BC, F), jnp.float32),
            pltpu.SemaphoreType.DMA((nbuf,)),
            pltpu.SemaphoreType.DMA((nbuf,)),
        ],
        compiler_params=_sc_params,
    )
    def agg_kernel(src_hbm, dst_hbm, z_hbm, zeros_hbm, out_hbm,
                   accum, srcbuf, dstbuf, rows, gsem, ssem):
        cid = lax.axis_index("c")
        sid = lax.axis_index("s")

        @pl.when(cid == 0)
        def _():
            base = sid * C0
            pltpu.sync_copy(src_hbm.at[pl.ds(base, C0)], srcbuf)
            pltpu.sync_copy(dst_hbm.at[pl.ds(base, C0)], dstbuf)

        @pl.when(cid == 1)
        def _():
            base = NS * C0 + sid * C1
            pltpu.sync_copy(src_hbm.at[pl.ds(base, C1)], srcbuf.at[pl.ds(0, C1)])
            pltpu.sync_copy(dst_hbm.at[pl.ds(base, C1)], dstbuf.at[pl.ds(0, C1)])

        ngroups = jnp.where(cid == 0, C0 // nbuf, C1 // nbuf)

        def gather(i, b):
            pltpu.async_copy(z_hbm.at[srcbuf.at[i]], rows.at[b], gsem.at[b])

        def gather_wait(b):
            pltpu.make_async_copy(z_hbm.at[srcbuf.at[0]], rows.at[b], gsem.at[b]).wait()

        def scatter(i, b):
            pltpu.async_copy(rows.at[b], accum.at[dstbuf.at[i]], ssem.at[b], add=True)

        def scatter_wait(i, b):
            pltpu.make_async_copy(rows.at[b], accum.at[dstbuf.at[i]], ssem.at[b]).wait()

        for b in range(nbuf):
            gather(b, b)
        pltpu.sync_copy(zeros_hbm.at[pl.ds(0, ROWS_PT)], accum.at[pl.ds(sid * ROWS_PT, ROWS_PT)])
        plsc.subcore_barrier()

        def group(g, carry):
            i0 = g * nbuf
            for b in range(nbuf):
                gather_wait(b)
                scatter(i0 + b, b)
            for b in range(nbuf):
                scatter_wait(i0 + b, b)
                gather(i0 + nbuf + b, b)
            return carry

        lax.fori_loop(0, ngroups - 1, group, 0)
        i0 = (ngroups - 1) * nbuf
        for b in range(nbuf):
            gather_wait(b)
            scatter(i0 + b, b)
        for b in range(nbuf):
            scatter_wait(i0 + b, b)
        plsc.subcore_barrier()
        r0 = sid * ROWS_PT
        pltpu.sync_copy(accum.at[pl.ds(r0, ROWS_PT)], out_hbm.at[cid, pl.ds(r0, ROWS_PT)])

    return agg_kernel


def _tc1_body(degA, degB, x, W1, dis_o, z1_o):
    dis = lax.rsqrt(degA[...] + degB[...] + 1.0)
    dis_o[...] = dis
    z1_o[...] = dis * jnp.dot(x[...], W1[...], preferred_element_type=jnp.float32)


def _tc2_body(pA, pB, z1, dis, b1, z2_o):
    d = dis[...]
    a = d * (pA[...] + pB[...] + z1[...])
    z2_o[...] = d * jnp.maximum(a + b1[...], 0.0)


def _tc3_body(pA, pB, z2, dis, W2, b2, z3_o):
    d = dis[...]
    a = d * (pA[...] + pB[...] + z2[...])
    h = jnp.maximum(jnp.dot(a, W2[...], preferred_element_type=jnp.float32) + b2[...], 0.0)
    z3_o[...] = d * h


def _tc4_body(pA, pB, z3, dis, batch2, W3, b3, Wl, bl, out_o):
    a3 = dis[...] * (pA[...] + pB[...] + z3[...])
    seg = lax.broadcasted_iota(jnp.int32, (G, N), 0)
    onehot = (batch2[...] == seg).astype(jnp.float32)
    sums = jnp.dot(onehot, a3, preferred_element_type=jnp.float32)
    cnt = jnp.sum(onehot, axis=1, keepdims=True)
    pooled = sums / jnp.maximum(cnt, 1.0)
    head = jnp.dot(pooled, W3[...], preferred_element_type=jnp.float32) + b3[...]
    out_o[...] = jnp.dot(head, Wl[...], preferred_element_type=jnp.float32) + bl[...]


def _tc_call(body, out_shapes):
    return pl.pallas_call(body, out_shape=out_shapes)


def kernel(x, edge_index, batch, W1, b1, W2, b2, W3, b3, Wl, bl):
    E = edge_index.shape[1]

    PER_W, CHUNKS, E_PAD = _edge_layout(E)
    pad = E_PAD - E
    src_f = jnp.concatenate([edge_index[0], jnp.zeros((pad,), jnp.int32)])
    dst_f = jnp.concatenate([edge_index[1], jnp.full((pad,), N, jnp.int32)])
    dst_w128 = dst_f.reshape(NW, PER_W // 128, 128)
    src_c128 = src_f.reshape(-1, 128)
    dst_c128 = dst_f.reshape(-1, 128)
    src_c64 = src_f.reshape(-1, 64)
    dst_c64 = dst_f.reshape(-1, 64)

    zeros8 = jnp.zeros((ROWS_PT, 8), jnp.float32)
    ones8 = jnp.ones((B, 8), jnp.float32)
    zeros64 = jnp.zeros((ROWS_PT, H), jnp.float32)
    zeros128 = jnp.zeros((ROWS_PT, 2 * H), jnp.float32)

    deg_parts = _make_deg_kernel(E)(dst_w128, zeros8, ones8)
    degA = deg_parts[0, :N, 0:1]
    degB = deg_parts[1, :N, 0:1]

    agg64 = _make_agg_kernel(E, H, 128, 32)
    agg128 = _make_agg_kernel(E, 2 * H, 230, 90)

    dis, z1 = _tc_call(_tc1_body, (
        jax.ShapeDtypeStruct((N, 1), jnp.float32),
        jax.ShapeDtypeStruct((N, H), jnp.float32),
    ))(degA, degB, x, W1)

    s1 = agg64(src_c128, dst_c128, z1, zeros64)
    z2 = _tc_call(_tc2_body, jax.ShapeDtypeStruct((N, H), jnp.float32))(
        s1[0, :N, :], s1[1, :N, :], z1, dis, b1.reshape(1, H))

    s2 = agg64(src_c128, dst_c128, z2, zeros64)
    z3 = _tc_call(_tc3_body, jax.ShapeDtypeStruct((N, 2 * H), jnp.float32))(
        s2[0, :N, :], s2[1, :N, :], z2, dis, W2, b2.reshape(1, 2 * H))

    s3 = agg128(src_c64, dst_c64, z3, zeros128)
    out = _tc_call(_tc4_body, jax.ShapeDtypeStruct((G, 1), jnp.float32))(
        s3[0, :N, :], s3[1, :N, :], z3, dis, batch.reshape(1, N),
        W3, b3.reshape(1, 2 * H), Wl, bl.reshape(1, 1))
    return out


# speedup vs baseline: 1.0897x; 1.0000x over previous
import functools

import jax
import jax.numpy as jnp
from jax import lax
from jax.experimental import pallas as pl
from jax.experimental.pallas import tpu as pltpu
from jax.experimental.pallas import tpu_sc as plsc

N = 10000
D = 128
H = 64
G = 64

NC = 2
NS = 16
NW = NC * NS

B = 128
N_PAD = NS * 640
ROWS_PT = N_PAD // NS

_mesh = plsc.VectorSubcoreMesh(core_axis_name="c", subcore_axis_name="s")
_sc_params = pltpu.CompilerParams(use_tc_tiling_on_sc=False)


def _edge_layout(E):
    q = B * NBUF
    per_w = ((E + NW - 1) // NW + q - 1) // q * q
    return per_w, per_w // B, NW * per_w


NBUF = 4


@functools.lru_cache(maxsize=None)
def _make_deg_kernel(E):
    PER_W, CHUNKS, _ = _edge_layout(E)

    @functools.partial(
        pl.kernel,
        out_type=jax.ShapeDtypeStruct((NC, N_PAD, 8), jnp.float32),
        mesh=_mesh,
        scratch_types=[
            pltpu.VMEM_SHARED((N_PAD, 8), jnp.float32),
            pltpu.VMEM((CHUNKS, B), jnp.int32),
            pltpu.VMEM((B, 8), jnp.float32),
            pltpu.SemaphoreType.DMA((NBUF,)),
        ],
        compiler_params=_sc_params,
    )
    def deg_kernel(dst_hbm, zeros_hbm, ones_hbm, out_hbm, accum, dstbuf, ones_v, sems):
        cid = lax.axis_index("c")
        sid = lax.axis_index("s")
        wid = cid * NS + sid
        pltpu.sync_copy(dst_hbm.at[wid], dstbuf)
        pltpu.sync_copy(zeros_hbm.at[pl.ds(0, ROWS_PT)], accum.at[pl.ds(sid * ROWS_PT, ROWS_PT)])
        pltpu.sync_copy(ones_hbm, ones_v)
        plsc.subcore_barrier()

        def group(g, carry):
            for b in range(NBUF):
                pltpu.async_copy(ones_v, accum.at[dstbuf.at[g * NBUF + b]],
                                 sems.at[b], add=True)
            for b in range(NBUF):
                pltpu.make_async_copy(ones_v, accum.at[dstbuf.at[g * NBUF + b]],
                                      sems.at[b]).wait()
            return carry

        lax.fori_loop(0, CHUNKS // NBUF, group, 0)
        plsc.subcore_barrier()
        r0 = sid * ROWS_PT
        pltpu.sync_copy(accum.at[pl.ds(r0, ROWS_PT)], out_hbm.at[cid, pl.ds(r0, ROWS_PT)])

    return deg_kernel


@functools.lru_cache(maxsize=None)
def _make_agg_kernel(E, F, C0, C1):
    PER_W, _, _ = _edge_layout(E)
    BC = 8192 // F
    TOT_CHUNKS = NW * PER_W // BC
    nbuf = 4 if F <= 64 else 2
    assert NS * (C0 + C1) == TOT_CHUNKS
    assert C0 % nbuf == 0 and C1 % nbuf == 0 and C1 // nbuf >= 2

    @functools.partial(
        pl.kernel,
        out_type=jax.ShapeDtypeStruct((NC, N_PAD, F), jnp.float32),
        mesh=_mesh,
        scratch_types=[
            pltpu.VMEM_SHARED((N_PAD, F), jnp.float32),
            pltpu.VMEM((C0, BC), jnp.int32),
            pltpu.VMEM((C0, BC), jnp.int32),
            pltpu.VMEM((nbuf, BC, F), jnp.float32),
            pltpu.SemaphoreType.DMA((nbuf,)),
            pltpu.SemaphoreType.DMA((nbuf,)),
        ],
        compiler_params=_sc_params,
    )
    def agg_kernel(src_hbm, dst_hbm, z_hbm, zeros_hbm, out_hbm,
                   accum, srcbuf, dstbuf, rows, gsem, ssem):
        cid = lax.axis_index("c")
        sid = lax.axis_index("s")

        with jax.named_scope("idx_preload"):
            @pl.when(cid == 0)
            def _():
                base = sid * C0
                pltpu.sync_copy(src_hbm.at[pl.ds(base, C0)], srcbuf)
                pltpu.sync_copy(dst_hbm.at[pl.ds(base, C0)], dstbuf)

            @pl.when(cid == 1)
            def _():
                base = NS * C0 + sid * C1
                pltpu.sync_copy(src_hbm.at[pl.ds(base, C1)], srcbuf.at[pl.ds(0, C1)])
                pltpu.sync_copy(dst_hbm.at[pl.ds(base, C1)], dstbuf.at[pl.ds(0, C1)])

        ngroups = jnp.where(cid == 0, C0 // nbuf, C1 // nbuf)

        def gather(i, b):
            pltpu.async_copy(z_hbm.at[srcbuf.at[i]], rows.at[b], gsem.at[b])

        def gather_wait(b):
            pltpu.make_async_copy(z_hbm.at[srcbuf.at[0]], rows.at[b], gsem.at[b]).wait()

        def scatter(i, b):
            pltpu.async_copy(rows.at[b], accum.at[dstbuf.at[i]], ssem.at[b], add=True)

        def scatter_wait(i, b):
            pltpu.make_async_copy(rows.at[b], accum.at[dstbuf.at[i]], ssem.at[b]).wait()

        with jax.named_scope("zero_prime"):
            for b in range(nbuf):
                gather(b, b)
            pltpu.sync_copy(zeros_hbm.at[pl.ds(0, ROWS_PT)], accum.at[pl.ds(sid * ROWS_PT, ROWS_PT)])
            plsc.subcore_barrier()

        def group(g, carry):
            i0 = g * nbuf
            for b in range(nbuf):
                gather_wait(b)
                scatter(i0 + b, b)
            for b in range(nbuf):
                scatter_wait(i0 + b, b)
                gather(i0 + nbuf + b, b)
            return carry

        with jax.named_scope("edge_loop"):
            lax.fori_loop(0, ngroups - 1, group, 0)
            i0 = (ngroups - 1) * nbuf
            for b in range(nbuf):
                gather_wait(b)
                scatter(i0 + b, b)
            for b in range(nbuf):
                scatter_wait(i0 + b, b)
        with jax.named_scope("drain_out"):
            plsc.subcore_barrier()
            r0 = sid * ROWS_PT
            pltpu.sync_copy(accum.at[pl.ds(r0, ROWS_PT)], out_hbm.at[cid, pl.ds(r0, ROWS_PT)])

    return agg_kernel


def _tc1_body(degA, degB, x, W1, dis_o, z1_o):
    dis = lax.rsqrt(degA[...] + degB[...] + 1.0)
    dis_o[...] = dis
    z1_o[...] = dis * jnp.dot(x[...], W1[...], preferred_element_type=jnp.float32)


def _tc2_body(pA, pB, z1, dis, b1, z2_o):
    d = dis[...]
    a = d * (pA[...] + pB[...] + z1[...])
    z2_o[...] = d * jnp.maximum(a + b1[...], 0.0)


def _tc3_body(pA, pB, z2, dis, W2, b2, z3_o):
    d = dis[...]
    a = d * (pA[...] + pB[...] + z2[...])
    h = jnp.maximum(jnp.dot(a, W2[...], preferred_element_type=jnp.float32) + b2[...], 0.0)
    z3_o[...] = d * h


def _tc4_body(pA, pB, z3, dis, batch2, W3, b3, Wl, bl, out_o):
    a3 = dis[...] * (pA[...] + pB[...] + z3[...])
    seg = lax.broadcasted_iota(jnp.int32, (G, N), 0)
    onehot = (batch2[...] == seg).astype(jnp.float32)
    sums = jnp.dot(onehot, a3, preferred_element_type=jnp.float32)
    cnt = jnp.sum(onehot, axis=1, keepdims=True)
    pooled = sums / jnp.maximum(cnt, 1.0)
    head = jnp.dot(pooled, W3[...], preferred_element_type=jnp.float32) + b3[...]
    out_o[...] = jnp.dot(head, Wl[...], preferred_element_type=jnp.float32) + bl[...]


def _tc_call(body, out_shapes):
    return pl.pallas_call(body, out_shape=out_shapes)


def kernel(x, edge_index, batch, W1, b1, W2, b2, W3, b3, Wl, bl):
    E = edge_index.shape[1]

    PER_W, CHUNKS, E_PAD = _edge_layout(E)
    pad = E_PAD - E
    src_f = jnp.concatenate([edge_index[0], jnp.zeros((pad,), jnp.int32)])
    dst_f = jnp.concatenate([edge_index[1], jnp.full((pad,), N, jnp.int32)])
    dst_w128 = dst_f.reshape(NW, PER_W // 128, 128)
    src_c128 = src_f.reshape(-1, 128)
    dst_c128 = dst_f.reshape(-1, 128)
    src_c64 = src_f.reshape(-1, 64)
    dst_c64 = dst_f.reshape(-1, 64)

    zeros8 = jnp.zeros((ROWS_PT, 8), jnp.float32)
    ones8 = jnp.ones((B, 8), jnp.float32)
    zeros64 = jnp.zeros((ROWS_PT, H), jnp.float32)
    zeros128 = jnp.zeros((ROWS_PT, 2 * H), jnp.float32)

    deg_parts = _make_deg_kernel(E)(dst_w128, zeros8, ones8)
    degA = deg_parts[0, :N, 0:1]
    degB = deg_parts[1, :N, 0:1]

    agg64 = _make_agg_kernel(E, H, 128, 32)
    agg128 = _make_agg_kernel(E, 2 * H, 230, 90)

    dis, z1 = _tc_call(_tc1_body, (
        jax.ShapeDtypeStruct((N, 1), jnp.float32),
        jax.ShapeDtypeStruct((N, H), jnp.float32),
    ))(degA, degB, x, W1)

    s1 = agg64(src_c128, dst_c128, z1, zeros64)
    z2 = _tc_call(_tc2_body, jax.ShapeDtypeStruct((N, H), jnp.float32))(
        s1[0, :N, :], s1[1, :N, :], z1, dis, b1.reshape(1, H))

    s2 = agg64(src_c128, dst_c128, z2, zeros64)
    z3 = _tc_call(_tc3_body, jax.ShapeDtypeStruct((N, 2 * H), jnp.float32))(
        s2[0, :N, :], s2[1, :N, :], z2, dis, W2, b2.reshape(1, 2 * H))

    s3 = agg128(src_c64, dst_c64, z3, zeros128)
    out = _tc_call(_tc4_body, jax.ShapeDtypeStruct((G, 1), jnp.float32))(
        s3[0, :N, :], s3[1, :N, :], z3, dis, batch.reshape(1, N),
        W3, b3.reshape(1, 2 * H), Wl, bl.reshape(1, 1))
    return out
